# baseline (speedup 1.0000x reference)
import functools

import jax
import jax.numpy as jnp
from jax import lax
from jax.experimental import pallas as pl
from jax.experimental.pallas import tpu as pltpu
from jax.experimental.pallas import tpu_sc as plsc

_NC = 2
_NS = 16
_NW = _NC * _NS

_C = 8


def _build_gather(BT, S, V, D):
    b_per_w = BT * S // _NW
    n_chunks = b_per_w // _C
    cpb = S // _C
    bt_per_w = BT // _NW
    assert n_chunks % 2 == 0
    mesh = plsc.VectorSubcoreMesh(core_axis_name="c", subcore_axis_name="s")

    @functools.partial(
        pl.kernel,
        mesh=mesh,
        out_type=jax.ShapeDtypeStruct((BT, S, D), jnp.float32),
        scratch_types=[
            pltpu.VMEM((n_chunks, _C), jnp.int32),
            pltpu.VMEM((_C, D), jnp.float32),
            pltpu.VMEM((_C, D), jnp.float32),
            pltpu.VMEM_SHARED((V, D), jnp.float32),
            pltpu.SemaphoreType.DMA,
            pltpu.SemaphoreType.DMA,
            pltpu.SemaphoreType.DMA,
            pltpu.SemaphoreType.DMA,
        ],
        compiler_params=pltpu.CompilerParams(use_tc_tiling_on_sc=False),
    )
    def gather_kernel(table_hbm, idx_hbm, out_hbm, idx_v,
                      rows0, rows1, spt, gsem0, gsem1, ssem0, ssem1):
        wid = lax.axis_index("s") * _NC + lax.axis_index("c")
        sid = lax.axis_index("s")
        bufs = ((rows0, gsem0, ssem0), (rows1, gsem1, ssem1))
        pltpu.sync_copy(idx_hbm.at[wid], idx_v)

        n_blk = V // 8
        for j in range((n_blk + _NS - 1) // _NS):
            blk = sid + _NS * j

            @pl.when(blk < n_blk)
            def _():
                pltpu.sync_copy(table_hbm.at[pl.ds(blk * 8, 8)],
                                spt.at[pl.ds(blk * 8, 8)])
        plsc.subcore_barrier()

        def out_block(i):
            b = wid * bt_per_w + i // cpb
            s0 = (i % cpb) * _C
            return out_hbm.at[b, pl.ds(s0, _C)]

        def wait_store(rb, ss):
            pltpu.make_async_copy(rb, out_hbm.at[0, pl.ds(0, _C)], ss).wait()

        def wait_gather(rb, gs):
            pltpu.make_async_copy(spt.at[idx_v.at[0]], rb, gs).wait()

        def pair(g, carry):
            for b, (rb, gs, ss) in enumerate(bufs):
                i = 2 * g + b

                @pl.when(g > 0)
                def _():
                    wait_store(rb, ss)

                pltpu.async_copy(spt.at[idx_v.at[i]], rb, gs)
            for b, (rb, gs, ss) in enumerate(bufs):
                i = 2 * g + b
                wait_gather(rb, gs)
                pltpu.async_copy(rb, out_block(i), ss)
            return carry

        lax.fori_loop(0, n_chunks // 2, pair, 0)
        for rb, gs, ss in bufs:
            wait_store(rb, ss)

    return gather_kernel


def kernel(x, table):
    BT, S = x.shape
    V, D = table.shape
    xf = x.reshape(_NW, BT * S // _NW // _C, _C).astype(jnp.int32)
    return _build_gather(BT, S, V, D)(table, xf)


# speedup vs baseline: 1.4903x; 1.4903x over previous
import functools

import jax
import jax.numpy as jnp
from jax import lax
from jax.experimental import pallas as pl
from jax.experimental.pallas import tpu as pltpu
from jax.experimental.pallas import tpu_sc as plsc

_NC = 2
_NS = 16
_NW = _NC * _NS

_C = 16
_L = 128
_V16 = 16


def _build_gather(B, V, D, DP):
    b_per_w = B // _NW
    n_chunks = b_per_w // _C
    n_rows = b_per_w // _L
    cpr = _L // _C
    mesh = plsc.VectorSubcoreMesh(core_axis_name="c", subcore_axis_name="s")

    @functools.partial(
        pl.kernel,
        mesh=mesh,
        out_type=jax.ShapeDtypeStruct((B, D), jnp.float32),
        scratch_types=[
            pltpu.VMEM((n_rows, _L), jnp.int32),
            pltpu.VMEM((_C, DP), jnp.float32),
            pltpu.VMEM((_C, DP), jnp.float32),
            pltpu.VMEM((_C, D), jnp.float32),
            pltpu.VMEM((_C, D), jnp.float32),
            pltpu.SemaphoreType.DMA,
            pltpu.SemaphoreType.DMA,
            pltpu.SemaphoreType.DMA,
            pltpu.SemaphoreType.DMA,
        ],
    )
    def gather_kernel(table_hbm, idx_hbm, out_hbm, idx_v,
                      raw0, raw1, cmp0, cmp1,
                      gsem0, gsem1, ssem0, ssem1):
        wid = lax.axis_index("s") * _NC + lax.axis_index("c")
        bufs = ((raw0, cmp0, gsem0, ssem0), (raw1, cmp1, gsem1, ssem1))
        pltpu.sync_copy(idx_hbm.at[wid], idx_v)

        def chunk_idx(i):
            return idx_v[i // cpr, pl.ds((i % cpr) * _C, _C)]

        def wait_store(cb, ss):
            pltpu.make_async_copy(cb, out_hbm.at[pl.ds(0, _C)], ss).wait()

        def wait_gather(rb, gs):
            pltpu.make_async_copy(table_hbm.at[chunk_idx(0)], rb, gs).wait()

        offs = list(range(0, D - _V16 + 1, _V16))
        if D % _V16:
            offs.append(D - _V16)

        def compact(rb, cb):
            def row(r, carry):
                for o in offs:
                    cb[r, pl.ds(o, _V16)] = rb[r, pl.ds(o, _V16)]
                return carry
            lax.fori_loop(0, _C, row, 0)

        def pair(g, carry):
            for b, (rb, cb, gs, ss) in enumerate(bufs):
                i = 2 * g + b

                @pl.when(g > 0)
                def _():
                    wait_store(cb, ss)

                pltpu.async_copy(table_hbm.at[chunk_idx(i)], rb, gs)
            for b, (rb, cb, gs, ss) in enumerate(bufs):
                i = 2 * g + b
                wait_gather(rb, gs)
                compact(rb, cb)
                row0 = (wid * n_chunks + i) * _C
                pltpu.async_copy(cb, out_hbm.at[pl.ds(row0, _C)], ss)
            return carry

        lax.fori_loop(0, n_chunks // 2, pair, 0)
        for rb, cb, gs, ss in bufs:
            wait_store(cb, ss)

    return gather_kernel


def kernel(x, table):
    B = x.shape[0] * x.shape[1]
    V, D = table.shape
    DP = (D + 127) // 128 * 128
    tablep = jnp.pad(table, ((0, 0), (0, DP - D)))
    xf = x.reshape(_NW, B // _NW // _L, _L).astype(jnp.int32)
    out = _build_gather(B, V, D, DP)(tablep, xf)
    return out.reshape(x.shape[0], x.shape[1], D)
